# speedup vs baseline: 1.4378x; 1.4378x over previous
import functools

import jax
import jax.numpy as jnp
from jax import lax
from jax.experimental import pallas as pl
from jax.experimental.pallas import tpu as pltpu
from jax.experimental.pallas import tpu_sc as plsc

_NC = 2
_NS = 16
_NW = _NC * _NS


@functools.partial(jax.jit, static_argnames=("chunk",))
def _sc_gather(idx, table, chunk=16):
    B = idx.shape[0]
    D = table.shape[1]
    b_per_w = B // _NW
    n_chunks = b_per_w // chunk
    mesh = plsc.VectorSubcoreMesh(core_axis_name="c", subcore_axis_name="s")

    @functools.partial(
        pl.kernel,
        out_type=jax.ShapeDtypeStruct((B, D), jnp.float32),
        mesh=mesh,
        scratch_types=[
            pltpu.VMEM((b_per_w,), jnp.int32),
            pltpu.VMEM((chunk, D), jnp.float32),
            pltpu.SemaphoreType.DMA,
        ],
    )
    def k(idx_hbm, table_hbm, out_hbm, idx_v, rows_v, sem):
        wid = lax.axis_index("s") * _NC + lax.axis_index("c")
        base = wid * b_per_w
        pltpu.sync_copy(idx_hbm.at[pl.ds(base, b_per_w)], idx_v)

        @pl.loop(0, n_chunks)
        def _(c):
            off = c * chunk
            pltpu.async_copy(
                table_hbm.at[idx_v.at[pl.ds(off, chunk)]], rows_v, sem
            ).wait()
            pltpu.sync_copy(rows_v, out_hbm.at[pl.ds(base + off, chunk)])

    return k(idx, table)


def kernel(input_ids, weight):
    b, s = input_ids.shape
    ids = input_ids.reshape(-1).astype(jnp.int32)
    out = _sc_gather(ids, weight)
    return out.reshape(b, s, weight.shape[1])


# speedup vs baseline: 1.5878x; 1.1043x over previous
import functools

import jax
import jax.numpy as jnp
from jax import lax
from jax.experimental import pallas as pl
from jax.experimental.pallas import tpu as pltpu
from jax.experimental.pallas import tpu_sc as plsc

_NC = 2
_NS = 16
_NW = _NC * _NS


@functools.partial(jax.jit, static_argnames=("chunk", "nbuf"))
def _sc_gather(idx, table, chunk=16, nbuf=2):
    B = idx.shape[0]
    D = table.shape[1]
    b_per_w = B // _NW
    n_chunks = b_per_w // chunk
    assert n_chunks % nbuf == 0
    mesh = plsc.VectorSubcoreMesh(core_axis_name="c", subcore_axis_name="s")

    @functools.partial(
        pl.kernel,
        out_type=jax.ShapeDtypeStruct((B, D), jnp.float32),
        mesh=mesh,
        scratch_types=(
            [pltpu.VMEM((b_per_w,), jnp.int32)]
            + [pltpu.VMEM((chunk, D), jnp.float32) for _ in range(nbuf)]
            + [pltpu.SemaphoreType.DMA for _ in range(2 * nbuf)]
        ),
    )
    def k(idx_hbm, table_hbm, out_hbm, idx_v, *bufs_and_sems):
        bufs = bufs_and_sems[:nbuf]
        gsems = bufs_and_sems[nbuf:2 * nbuf]
        osems = bufs_and_sems[2 * nbuf:]

        wid = lax.axis_index("s") * _NC + lax.axis_index("c")
        base = wid * b_per_w
        pltpu.sync_copy(idx_hbm.at[pl.ds(base, b_per_w)], idx_v)

        def start_gather(c, b):
            pltpu.make_async_copy(
                table_hbm.at[idx_v.at[pl.ds(c * chunk, chunk)]],
                bufs[b], gsems[b],
            ).start()

        def wait_gather(b):
            pltpu.make_async_copy(
                table_hbm.at[idx_v.at[pl.ds(0, chunk)]], bufs[b], gsems[b]
            ).wait()

        def start_out(c, b):
            pltpu.make_async_copy(
                bufs[b], out_hbm.at[pl.ds(base + c * chunk, chunk)], osems[b]
            ).start()

        def wait_out(b):
            pltpu.make_async_copy(
                bufs[b], out_hbm.at[pl.ds(base, chunk)], osems[b]
            ).wait()

        for b in range(nbuf):
            start_gather(b, b)

        @pl.loop(0, n_chunks, step=nbuf)
        def _(c):
            for b in range(nbuf):
                wait_gather(b)
                start_out(c + b, b)
            for b in range(nbuf):
                @pl.when(c + nbuf + b < n_chunks)
                def _(b=b):
                    wait_out(b)
                    start_gather(c + nbuf + b, b)

        for b in range(nbuf):
            wait_out(b)

    return k(idx, table)


def kernel(input_ids, weight):
    b, s = input_ids.shape
    ids = input_ids.reshape(-1).astype(jnp.int32)
    out = _sc_gather(ids, weight)
    return out.reshape(b, s, weight.shape[1])


# speedup vs baseline: 1.5932x; 1.0034x over previous
import functools

import jax
import jax.numpy as jnp
from jax import lax
from jax.experimental import pallas as pl
from jax.experimental.pallas import tpu as pltpu
from jax.experimental.pallas import tpu_sc as plsc

_NC = 2
_NS = 16
_NW = _NC * _NS


@functools.partial(jax.jit, static_argnames=("chunk", "nbuf"))
def _sc_gather(idx, table, chunk=8, nbuf=4):
    B = idx.shape[0]
    D = table.shape[1]
    b_per_w = B // _NW
    n_chunks = b_per_w // chunk
    assert n_chunks % nbuf == 0
    mesh = plsc.VectorSubcoreMesh(core_axis_name="c", subcore_axis_name="s")

    @functools.partial(
        pl.kernel,
        out_type=jax.ShapeDtypeStruct((B, D), jnp.float32),
        mesh=mesh,
        scratch_types=(
            [pltpu.VMEM((b_per_w,), jnp.int32)]
            + [pltpu.VMEM((chunk, D), jnp.float32) for _ in range(nbuf)]
            + [pltpu.SemaphoreType.DMA for _ in range(2 * nbuf)]
        ),
    )
    def k(idx_hbm, table_hbm, out_hbm, idx_v, *bufs_and_sems):
        bufs = bufs_and_sems[:nbuf]
        gsems = bufs_and_sems[nbuf:2 * nbuf]
        osems = bufs_and_sems[2 * nbuf:]

        wid = lax.axis_index("s") * _NC + lax.axis_index("c")
        base = wid * b_per_w
        pltpu.sync_copy(idx_hbm.at[pl.ds(base, b_per_w)], idx_v)

        def start_gather(c, b):
            pltpu.make_async_copy(
                table_hbm.at[idx_v.at[pl.ds(c * chunk, chunk)]],
                bufs[b], gsems[b],
            ).start()

        def wait_gather(b):
            pltpu.make_async_copy(
                table_hbm.at[idx_v.at[pl.ds(0, chunk)]], bufs[b], gsems[b]
            ).wait()

        def start_out(c, b):
            pltpu.make_async_copy(
                bufs[b], out_hbm.at[pl.ds(base + c * chunk, chunk)], osems[b]
            ).start()

        def wait_out(b):
            pltpu.make_async_copy(
                bufs[b], out_hbm.at[pl.ds(base, chunk)], osems[b]
            ).wait()

        for b in range(nbuf):
            start_gather(b, b)

        @pl.loop(0, n_chunks, step=nbuf)
        def _(c):
            for b in range(nbuf):
                wait_gather(b)
                start_out(c + b, b)
            for b in range(nbuf):
                @pl.when(c + nbuf + b < n_chunks)
                def _(b=b):
                    wait_out(b)
                    start_gather(c + nbuf + b, b)

        for b in range(nbuf):
            wait_out(b)

    return k(idx, table)


def kernel(input_ids, weight):
    b, s = input_ids.shape
    ids = input_ids.reshape(-1).astype(jnp.int32)
    out = _sc_gather(ids, weight)
    return out.reshape(b, s, weight.shape[1])


# speedup vs baseline: 1.6110x; 1.0111x over previous
import functools

import jax
import jax.numpy as jnp
from jax import lax
from jax.experimental import pallas as pl
from jax.experimental.pallas import tpu as pltpu
from jax.experimental.pallas import tpu_sc as plsc

_NC = 2
_NS = 16
_NW = _NC * _NS


@functools.partial(jax.jit, static_argnames=("chunk", "nbuf"))
def _sc_gather(idx, table, chunk=8, nbuf=4):
    S = idx.shape[1]
    B = idx.shape[0] * S
    D = table.shape[1]
    b_per_w = B // _NW
    n_chunks = b_per_w // chunk
    assert n_chunks % nbuf == 0
    mesh = plsc.VectorSubcoreMesh(core_axis_name="c", subcore_axis_name="s")

    @functools.partial(
        pl.kernel,
        out_type=jax.ShapeDtypeStruct((B, D), jnp.float32),
        mesh=mesh,
        scratch_types=(
            [pltpu.VMEM((b_per_w,), jnp.int32)]
            + [pltpu.VMEM((chunk, D), jnp.float32) for _ in range(nbuf)]
            + [pltpu.SemaphoreType.DMA for _ in range(2 * nbuf)]
        ),
    )
    def k(idx_hbm, table_hbm, out_hbm, idx_v, *bufs_and_sems):
        bufs = bufs_and_sems[:nbuf]
        gsems = bufs_and_sems[nbuf:2 * nbuf]
        osems = bufs_and_sems[2 * nbuf:]

        wid = lax.axis_index("s") * _NC + lax.axis_index("c")
        base = wid * b_per_w
        r = base // S
        cl = base - r * S
        pltpu.sync_copy(idx_hbm.at[r, pl.ds(cl, b_per_w)], idx_v)

        def start_gather(c, b):
            pltpu.make_async_copy(
                table_hbm.at[idx_v.at[pl.ds(c * chunk, chunk)]],
                bufs[b], gsems[b],
            ).start()

        def wait_gather(b):
            pltpu.make_async_copy(
                table_hbm.at[idx_v.at[pl.ds(0, chunk)]], bufs[b], gsems[b]
            ).wait()

        def start_out(c, b):
            pltpu.make_async_copy(
                bufs[b], out_hbm.at[pl.ds(base + c * chunk, chunk)], osems[b]
            ).start()

        def wait_out(b):
            pltpu.make_async_copy(
                bufs[b], out_hbm.at[pl.ds(base, chunk)], osems[b]
            ).wait()

        for b in range(nbuf):
            start_gather(b, b)

        @pl.loop(0, n_chunks, step=nbuf)
        def _(c):
            for b in range(nbuf):
                wait_gather(b)
                start_out(c + b, b)
            for b in range(nbuf):
                @pl.when(c + nbuf + b < n_chunks)
                def _(b=b):
                    wait_out(b)
                    start_gather(c + nbuf + b, b)

        for b in range(nbuf):
            wait_out(b)

    return k(idx, table)


def kernel(input_ids, weight):
    b, s = input_ids.shape
    out = _sc_gather(input_ids.astype(jnp.int32), weight)
    return out.reshape(b, s, weight.shape[1])


# speedup vs baseline: 1.6249x; 1.0086x over previous
import functools

import jax
import jax.numpy as jnp
from jax import lax
from jax.experimental import pallas as pl
from jax.experimental.pallas import tpu as pltpu
from jax.experimental.pallas import tpu_sc as plsc

_NC = 2
_NS = 16
_NW = _NC * _NS


@functools.partial(jax.jit, static_argnames=("chunk", "nbuf"))
def _sc_gather(idx, table, chunk=8, nbuf=4):
    S = idx.shape[1]
    B = idx.shape[0] * S
    D = table.shape[1]
    b_per_w = B // _NW
    n_chunks = b_per_w // chunk
    assert n_chunks % nbuf == 0
    mesh = plsc.VectorSubcoreMesh(core_axis_name="c", subcore_axis_name="s")

    @functools.partial(
        pl.kernel,
        out_type=jax.ShapeDtypeStruct((B, D), jnp.float32),
        mesh=mesh,
        scratch_types=(
            [pltpu.VMEM((b_per_w,), jnp.int32)]
            + [pltpu.VMEM((chunk, D), jnp.float32) for _ in range(nbuf)]
            + [pltpu.SemaphoreType.DMA for _ in range(2 * nbuf)]
        ),
    )
    def k(idx_hbm, table_hbm, out_hbm, idx_v, *bufs_and_sems):
        bufs = bufs_and_sems[:nbuf]
        gsems = bufs_and_sems[nbuf:2 * nbuf]
        osems = bufs_and_sems[2 * nbuf:]

        wid = lax.axis_index("c") * _NS + lax.axis_index("s")
        base = wid * b_per_w
        r = base // S
        cl = base - r * S
        pltpu.sync_copy(idx_hbm.at[r, pl.ds(cl, b_per_w)], idx_v)

        def start_gather(c, b):
            pltpu.make_async_copy(
                table_hbm.at[idx_v.at[pl.ds(c * chunk, chunk)]],
                bufs[b], gsems[b],
            ).start()

        def wait_gather(b):
            pltpu.make_async_copy(
                table_hbm.at[idx_v.at[pl.ds(0, chunk)]], bufs[b], gsems[b]
            ).wait()

        def start_out(c, b):
            pltpu.make_async_copy(
                bufs[b], out_hbm.at[pl.ds(base + c * chunk, chunk)], osems[b]
            ).start()

        def wait_out(b):
            pltpu.make_async_copy(
                bufs[b], out_hbm.at[pl.ds(base, chunk)], osems[b]
            ).wait()

        for b in range(nbuf):
            start_gather(b, b)

        @pl.loop(0, n_chunks, step=nbuf)
        def _(c):
            for b in range(nbuf):
                wait_gather(b)
                start_out(c + b, b)
            for b in range(nbuf):
                @pl.when(c + nbuf + b < n_chunks)
                def _(b=b):
                    wait_out(b)
                    start_gather(c + nbuf + b, b)

        for b in range(nbuf):
            wait_out(b)

    return k(idx, table)


def kernel(input_ids, weight):
    b, s = input_ids.shape
    out = _sc_gather(input_ids.astype(jnp.int32), weight)
    return out.reshape(b, s, weight.shape[1])
